# baseline (speedup 1.0000x reference)
import functools

import jax
import jax.numpy as jnp
from jax import lax
from jax.experimental import pallas as pl
from jax.experimental.pallas import tpu as pltpu
from jax.experimental.pallas import tpu_sc as plsc

_BATCH, _SEQ, _EMBED = 4096, 50, 64
_NC, _NS = 2, 16
_NW = _NC * _NS
_BPW = _BATCH // _NW
_LANES = 16
_COLS = _EMBED // _LANES
_UNROLL = 4
_INV_SEQ = 1.0 / _SEQ


def _make_cbow():
  mesh = plsc.VectorSubcoreMesh(
      core_axis_name="c", subcore_axis_name="s",
      num_cores=_NC, num_subcores=_NS)

  @functools.partial(
      pl.kernel,
      mesh=mesh,
      compiler_params=pltpu.CompilerParams(use_tc_tiling_on_sc=False),
      out_type=jax.ShapeDtypeStruct((_BATCH, _EMBED), jnp.float32),
      scratch_types=[
          pltpu.VMEM((_SEQ, _BPW), jnp.int32),
          pltpu.VMEM((_BPW, _EMBED), jnp.float32),
          pltpu.VMEM((_BPW, _EMBED), jnp.float32),
          pltpu.VMEM((_BPW, _EMBED), jnp.float32),
          pltpu.SemaphoreType.DMA,
          pltpu.SemaphoreType.DMA,
      ],
  )
  def cbow(xr_hbm, emb_hbm, out_hbm, idx_v, rows0, rows1, acc, sem0, sem1):
    wid = lax.axis_index("s") * _NC + lax.axis_index("c")
    row0 = wid * _BPW

    pltpu.sync_copy(xr_hbm.at[wid], idx_v)

    rows = (rows0, rows1)
    sems = (sem0, sem1)

    pending = pltpu.async_copy(emb_hbm.at[idx_v.at[0]], rows0, sem0)

    for s in range(_SEQ):
      b = s & 1
      pending.wait()
      if s + 1 < _SEQ:
        pending = pltpu.async_copy(
            emb_hbm.at[idx_v.at[s + 1]], rows[1 - b], sems[1 - b])
      src = rows[b]

      if s == 0:
        def init_body(i, _):
          r = i * _UNROLL
          for d in range(_UNROLL):
            for c in range(_COLS):
              acc[r + d, pl.ds(c * _LANES, _LANES)] = (
                  src[r + d, pl.ds(c * _LANES, _LANES)])
          return 0
        lax.fori_loop(0, _BPW // _UNROLL, init_body, 0)
      else:
        def acc_body(i, _, src=src):
          r = i * _UNROLL
          for d in range(_UNROLL):
            for c in range(_COLS):
              plsc.addupdate(
                  acc.at[r + d, pl.ds(c * _LANES, _LANES)],
                  src[r + d, pl.ds(c * _LANES, _LANES)])
          return 0
        lax.fori_loop(0, _BPW // _UNROLL, acc_body, 0)

    def scale_body(i, _):
      r = i * _UNROLL
      for d in range(_UNROLL):
        for c in range(_COLS):
          sl = pl.ds(c * _LANES, _LANES)
          acc[r + d, sl] = acc[r + d, sl] * _INV_SEQ
      return 0
    lax.fori_loop(0, _BPW // _UNROLL, scale_body, 0)

    pltpu.sync_copy(acc, out_hbm.at[pl.ds(row0, _BPW)])

  return cbow


_cbow = _make_cbow()


@jax.jit
def kernel(X, emb):
  xr = X.astype(jnp.int32).reshape(_NW, _BPW, _SEQ).transpose(0, 2, 1)
  return _cbow(xr, emb)
